# baseline (speedup 1.0000x reference)
import functools

import jax
import jax.numpy as jnp
from jax import lax
from jax.experimental import pallas as pl
from jax.experimental.pallas import tpu as pltpu
from jax.experimental.pallas import tpu_sc as plsc

_CB = 256


def _select_body(judge_ref, xt_ref, c_ref, idx_ref, iflat_ref):
    b = pl.program_id(0)
    cb = pl.program_id(1)
    n = xt_ref.shape[2]
    judge = judge_ref[...]

    px = xt_ref[0, 0:1, :]
    py = xt_ref[0, 1:2, :]
    pz = xt_ref[0, 2:3, :]
    c = c_ref[0]
    dx = px - c[:, 0:1]
    dy = py - c[:, 1:2]
    dz = pz - c[:, 2:3]
    dist = dx * dx + dy * dy + dz * dz

    lane = lax.broadcasted_iota(jnp.int32, (_CB, n), 1)
    cid = cb * _CB + lax.broadcasted_iota(jnp.int32, (_CB, n), 0)
    base = jnp.where(lane == cid, judge, jnp.float32(1e10))
    valid = (dist > 1e-10) & (dist < judge)
    bx = dx >= 0.0
    by = dy >= 0.0
    bz = dz >= 0.0
    nbx = jnp.logical_not(bx)
    nby = jnp.logical_not(by)
    nbz = jnp.logical_not(bz)
    mx = [valid & nbx, valid & bx]
    mxy = [mx[0] & nby, mx[0] & by, mx[1] & nby, mx[1] & by]
    m8 = []
    for q in mxy:
        m8.append(q & nbz)
        m8.append(q & bz)

    cols = []
    for i in range(8):
        di = jnp.where(m8[i], dist, base)
        mv = jnp.min(di, axis=1, keepdims=True)
        ii = jnp.min(jnp.where(di == mv, lane, n), axis=1, keepdims=True)
        cols.append(ii)
    idx = jnp.concatenate(cols, axis=1)
    idx_ref[0] = idx
    iflat_ref[0] = idx + b * n


def _tc_select(judge, xyz_t, xyz):
    b, n, _ = xyz.shape
    grid = (b, n // _CB)
    return pl.pallas_call(
        _select_body,
        grid=grid,
        in_specs=[
            pl.BlockSpec((1, 1), lambda bi, ci: (0, 0)),
            pl.BlockSpec((1, 3, n), lambda bi, ci: (bi, 0, 0)),
            pl.BlockSpec((1, _CB, 3), lambda bi, ci: (bi, ci, 0)),
        ],
        out_specs=[
            pl.BlockSpec((1, _CB, 8), lambda bi, ci: (bi, ci, 0)),
            pl.BlockSpec((1, _CB, 8), lambda bi, ci: (bi, ci, 0)),
        ],
        out_shape=[
            jax.ShapeDtypeStruct((b, n, 8), jnp.int32),
            jax.ShapeDtypeStruct((b, n, 8), jnp.int32),
        ],
    )(judge, xyz_t, xyz)


def _gather_body(nc, cpw, dp, table_hbm, xyz16_hbm, iflat_hbm,
                 pa_hbm, pb_hbm, gx_hbm,
                 idxv0, idxv1, rows0, rows1, cbuf0, cbuf1, gxb0, gxb1,
                 sg0, sg1, sw0, sw1):
    wid = lax.axis_index("s") * nc + lax.axis_index("c")
    wbase = wid * cpw
    lane16 = lax.iota(jnp.int32, 16)
    mask3 = lane16 < 3
    npair = cpw // 32

    def compute(rows, cbuf, gxb):
        cvecs = []
        for t in range(16):
            raw = cbuf[t, pl.ds(0, 16)]
            cvecs.append(jnp.where(mask3, raw, jnp.float32(0.0)))
        tail = (lane16 >= 8) & (lane16 < 11)
        for r in range(128):
            rv = jnp.full((16,), r, jnp.int32)
            v = rows[r, pl.ds(0, 16)] - cvecs[r // 8]
            rows[r, pl.ds(0, 16)] = v
            plsc.store_scatter(gxb, [rv, lane16], v, mask=mask3)
            v2 = rows[r, pl.ds(248, 16)]
            plsc.store_scatter(gxb, [rv, lane16 - 5], v2, mask=tail)

    def pair_body(g, carry):
        slots = (
            (2 * g, idxv0, rows0, cbuf0, gxb0, sg0, sw0),
            (2 * g + 1, idxv1, rows1, cbuf1, gxb1, sg1, sw1),
        )
        gathers = []
        for ch, idxv, rows, cbuf, _, sg, _ in slots:
            cbase = wbase + ch * 16
            pltpu.sync_copy(iflat_hbm.at[pl.ds(cbase * 8, 128)], idxv)
            pltpu.sync_copy(xyz16_hbm.at[pl.ds(cbase, 16)], cbuf)
            gathers.append(pltpu.async_copy(table_hbm.at[idxv], rows, sg))
        writes = []
        for i, (ch, idxv, rows, cbuf, gxb, sg, sw) in enumerate(slots):
            cbase = wbase + ch * 16
            rowbase = cbase * 8
            gathers[i].wait()
            compute(rows, cbuf, gxb)
            writes.append(pltpu.async_copy(
                rows.at[:, pl.ds(0, 128)],
                pa_hbm.at[pl.ds(rowbase, 128)], sw))
            writes.append(pltpu.async_copy(
                rows.at[:, pl.ds(128, 128)],
                pb_hbm.at[pl.ds(rowbase, 128)], sw))
            writes.append(
                pltpu.async_copy(gxb, gx_hbm.at[pl.ds(rowbase, 128)], sw))
        for w in writes:
            w.wait()
        return carry

    lax.fori_loop(0, npair, pair_body, 0)


def _sc_gather(table, xyz16, iflat):
    bn, dp = table.shape
    info = plsc.get_sparse_core_info()
    nw = info.num_cores * info.num_subcores
    cpw = bn // nw
    mesh = plsc.VectorSubcoreMesh(core_axis_name="c", subcore_axis_name="s")
    return pl.kernel(
        functools.partial(_gather_body, info.num_cores, cpw, dp),
        out_type=(
            jax.ShapeDtypeStruct((bn * 8, 128), jnp.float32),
            jax.ShapeDtypeStruct((bn * 8, 128), jnp.float32),
            jax.ShapeDtypeStruct((bn * 8, 8), jnp.float32),
        ),
        mesh=mesh,
        compiler_params=pltpu.CompilerParams(
            use_tc_tiling_on_sc=False, needs_layout_passes=False),
        scratch_types=[
            pltpu.VMEM((128,), jnp.int32),
            pltpu.VMEM((128,), jnp.int32),
            pltpu.VMEM((128, dp), jnp.float32),
            pltpu.VMEM((128, dp), jnp.float32),
            pltpu.VMEM((16, 16), jnp.float32),
            pltpu.VMEM((16, 16), jnp.float32),
            pltpu.VMEM((128, 8), jnp.float32),
            pltpu.VMEM((128, 8), jnp.float32),
            pltpu.SemaphoreType.DMA,
            pltpu.SemaphoreType.DMA,
            pltpu.SemaphoreType.DMA,
            pltpu.SemaphoreType.DMA,
        ],
    )(table, xyz16, iflat)


def kernel(radius, xyz, points):
    b, n, _ = xyz.shape
    cp = points.shape[-1]
    d = cp + 3
    dp = -(-d // 8) * 8
    judge = (jnp.asarray(radius, jnp.float32) ** 2).reshape(1, 1)
    xyz_t = jnp.transpose(xyz, (0, 2, 1))
    idx, iflat = _tc_select(judge, xyz_t, xyz)
    table = jnp.concatenate(
        [xyz, points, jnp.zeros((b, n, dp - d), jnp.float32)],
        axis=-1).reshape(b * n, dp)
    xyz16 = jnp.concatenate(
        [xyz, jnp.zeros((b, n, 13), jnp.float32)], axis=-1).reshape(b * n, 16)
    pa, pb, gx = _sc_gather(table, xyz16, iflat.reshape(b * n * 8))
    gx4 = jnp.reshape(gx, (b, n, 8, 8))
    grouped_xyz = gx4[..., :3]
    grouped_points = jnp.concatenate(
        [jnp.reshape(pa, (b, n, 8, 128)),
         jnp.reshape(pb, (b, n, 8, 128)),
         gx4[..., 3:3 + d - 256]],
        axis=-1)
    return (grouped_xyz, grouped_points, idx)


# speedup vs baseline: 1.0588x; 1.0588x over previous
import functools

import jax
import jax.numpy as jnp
from jax import lax
from jax.experimental import pallas as pl
from jax.experimental.pallas import tpu as pltpu
from jax.experimental.pallas import tpu_sc as plsc

_CB = 256


def _select_body(judge_ref, xt_ref, c_ref, idx_ref, iflat_ref):
    b = pl.program_id(0)
    cb = pl.program_id(1)
    n = xt_ref.shape[2]
    judge = judge_ref[...]

    px = xt_ref[0, 0:1, :]
    py = xt_ref[0, 1:2, :]
    pz = xt_ref[0, 2:3, :]
    c = c_ref[0]
    dx = px - c[:, 0:1]
    dy = py - c[:, 1:2]
    dz = pz - c[:, 2:3]
    dist = dx * dx + dy * dy + dz * dz

    lane = lax.broadcasted_iota(jnp.int32, (_CB, n), 1)
    cid = cb * _CB + lax.broadcasted_iota(jnp.int32, (_CB, n), 0)
    base = jnp.where(lane == cid, judge, jnp.float32(1e10))
    valid = (dist > 1e-10) & (dist < judge)
    bx = dx >= 0.0
    by = dy >= 0.0
    bz = dz >= 0.0
    nbx = jnp.logical_not(bx)
    nby = jnp.logical_not(by)
    nbz = jnp.logical_not(bz)
    mx = [valid & nbx, valid & bx]
    mxy = [mx[0] & nby, mx[0] & by, mx[1] & nby, mx[1] & by]
    m8 = []
    for q in mxy:
        m8.append(q & nbz)
        m8.append(q & bz)

    cols = []
    for i in range(8):
        di = jnp.where(m8[i], dist, base)
        mv = jnp.min(di, axis=1, keepdims=True)
        ii = jnp.min(jnp.where(di == mv, lane, n), axis=1, keepdims=True)
        cols.append(ii)
    idx = jnp.concatenate(cols, axis=1)
    idx_ref[0] = idx
    iflat_ref[0] = idx + b * n


def _tc_select(judge, xyz_t, xyz):
    b, n, _ = xyz.shape
    grid = (b, n // _CB)
    return pl.pallas_call(
        _select_body,
        grid=grid,
        compiler_params=pltpu.CompilerParams(
            dimension_semantics=("parallel", "parallel")),
        in_specs=[
            pl.BlockSpec((1, 1), lambda bi, ci: (0, 0)),
            pl.BlockSpec((1, 3, n), lambda bi, ci: (bi, 0, 0)),
            pl.BlockSpec((1, _CB, 3), lambda bi, ci: (bi, ci, 0)),
        ],
        out_specs=[
            pl.BlockSpec((1, _CB, 8), lambda bi, ci: (bi, ci, 0)),
            pl.BlockSpec((1, _CB, 8), lambda bi, ci: (bi, ci, 0)),
        ],
        out_shape=[
            jax.ShapeDtypeStruct((b, n, 8), jnp.int32),
            jax.ShapeDtypeStruct((b, n, 8), jnp.int32),
        ],
    )(judge, xyz_t, xyz)


def _gather_body(nc, cpw, dp, table_hbm, xyz16_hbm, iflat_hbm,
                 pa_hbm, pb_hbm, pc_hbm, gx_hbm,
                 idxv0, idxv1, rows0, rows1, cbuf0, cbuf1, gxb0, gxb1,
                 sg0, sg1, sw0, sw1):
    wid = lax.axis_index("s") * nc + lax.axis_index("c")
    wbase = wid * cpw
    lane16 = lax.iota(jnp.int32, 16)
    mask3 = lane16 < 3
    npair = cpw // 32

    def compute(rows, cbuf, gxb):
        cvecs = []
        for t in range(16):
            raw = cbuf[t, pl.ds(0, 16)]
            cvecs.append(jnp.where(mask3, raw, jnp.float32(0.0)))
        for r in range(128):
            v = rows[r, pl.ds(0, 16)] - cvecs[r // 8]
            rows[r, pl.ds(0, 16)] = v
            plsc.store_scatter(
                gxb, [jnp.full((16,), r, jnp.int32), lane16], v, mask=mask3)

    def pair_body(g, carry):
        slots = (
            (2 * g, idxv0, rows0, cbuf0, gxb0, sg0, sw0),
            (2 * g + 1, idxv1, rows1, cbuf1, gxb1, sg1, sw1),
        )
        gathers = []
        for ch, idxv, rows, cbuf, _, sg, _ in slots:
            cbase = wbase + ch * 16
            pltpu.sync_copy(iflat_hbm.at[pl.ds(cbase * 8, 128)], idxv)
            pltpu.sync_copy(xyz16_hbm.at[pl.ds(cbase, 16)], cbuf)
            gathers.append(pltpu.async_copy(table_hbm.at[idxv], rows, sg))
        writes = []
        for i, (ch, idxv, rows, cbuf, gxb, sg, sw) in enumerate(slots):
            cbase = wbase + ch * 16
            rowbase = cbase * 8
            gathers[i].wait()
            compute(rows, cbuf, gxb)
            writes.append(pltpu.async_copy(
                rows.at[:, pl.ds(0, 128)],
                pa_hbm.at[pl.ds(rowbase, 128)], sw))
            writes.append(pltpu.async_copy(
                rows.at[:, pl.ds(128, 128)],
                pb_hbm.at[pl.ds(rowbase, 128)], sw))
            writes.append(pltpu.async_copy(
                rows.at[:, pl.ds(256, 8)],
                pc_hbm.at[pl.ds(rowbase, 128)], sw))
            writes.append(
                pltpu.async_copy(gxb, gx_hbm.at[pl.ds(rowbase, 128)], sw))
        for w in writes:
            w.wait()
        return carry

    lax.fori_loop(0, npair, pair_body, 0)


def _sc_gather(table, xyz16, iflat):
    bn, dp = table.shape
    info = plsc.get_sparse_core_info()
    nw = info.num_cores * info.num_subcores
    cpw = bn // nw
    mesh = plsc.VectorSubcoreMesh(core_axis_name="c", subcore_axis_name="s")
    return pl.kernel(
        functools.partial(_gather_body, info.num_cores, cpw, dp),
        out_type=(
            jax.ShapeDtypeStruct((bn * 8, 128), jnp.float32),
            jax.ShapeDtypeStruct((bn * 8, 128), jnp.float32),
            jax.ShapeDtypeStruct((bn * 8, 8), jnp.float32),
            jax.ShapeDtypeStruct((bn * 8, 8), jnp.float32),
        ),
        mesh=mesh,
        compiler_params=pltpu.CompilerParams(
            use_tc_tiling_on_sc=False, needs_layout_passes=False),
        scratch_types=[
            pltpu.VMEM((128,), jnp.int32),
            pltpu.VMEM((128,), jnp.int32),
            pltpu.VMEM((128, dp), jnp.float32),
            pltpu.VMEM((128, dp), jnp.float32),
            pltpu.VMEM((16, 16), jnp.float32),
            pltpu.VMEM((16, 16), jnp.float32),
            pltpu.VMEM((128, 8), jnp.float32),
            pltpu.VMEM((128, 8), jnp.float32),
            pltpu.SemaphoreType.DMA,
            pltpu.SemaphoreType.DMA,
            pltpu.SemaphoreType.DMA,
            pltpu.SemaphoreType.DMA,
        ],
    )(table, xyz16, iflat)


def kernel(radius, xyz, points):
    b, n, _ = xyz.shape
    cp = points.shape[-1]
    d = cp + 3
    dp = -(-d // 8) * 8
    judge = (jnp.asarray(radius, jnp.float32) ** 2).reshape(1, 1)
    xyz_t = jnp.transpose(xyz, (0, 2, 1))
    idx, iflat = _tc_select(judge, xyz_t, xyz)
    table = jnp.concatenate(
        [xyz, points, jnp.zeros((b, n, dp - d), jnp.float32)],
        axis=-1).reshape(b * n, dp)
    xyz16 = jnp.concatenate(
        [xyz, jnp.zeros((b, n, 13), jnp.float32)], axis=-1).reshape(b * n, 16)
    pa, pb, pc, gx = _sc_gather(table, xyz16, iflat.reshape(b * n * 8))
    grouped_xyz = jnp.reshape(gx, (b, n, 8, 8))[..., :3]
    grouped_points = jnp.concatenate(
        [jnp.reshape(pa, (b, n, 8, 128)),
         jnp.reshape(pb, (b, n, 8, 128)),
         jnp.reshape(pc, (b, n, 8, 8))[..., : d - 256]],
        axis=-1)
    return (grouped_xyz, grouped_points, idx)
